# baseline (speedup 1.0000x reference)
import jax
import jax.numpy as jnp
from jax import lax
from jax.experimental import pallas as pl
from jax.experimental.pallas import tpu as pltpu


def _conv_body(featp_ref, w1_ref, b1_ref, w2_ref, b2_ref, logit_ref):
    C = featp_ref.shape[1]
    CM = w1_ref.shape[1]
    HW = logit_ref.shape[2]
    W = 64
    PAD = 128
    TN = 512
    EXT = TN + 2 * PAD
    x = lax.broadcasted_iota(jnp.int32, (1, TN), 1) % W
    m_l = (x != 0).astype(jnp.float32)
    m_r = (x != W - 1).astype(jnp.float32)
    masks = (m_l, None, m_r)

    GROUPS = ((0, 1), (1, 2), (2, 3), (3, 4), (4, 6), (6, 7), (7, 8),
              (8, 10), (10, 11), (11, 12), (12, 14), (14, 15), (15, 16),
              (16, 17), (17, 18), (18, 19), (19, 21), (21, 22), (22, 23),
              (23, 25), (25, 26), (26, 27))

    def tile_body(t, _):
        n0 = t * TN
        ext = featp_ref[0, :, pl.ds(pl.multiple_of(n0, 128), EXT)]
        bts = []
        for dyi, dy in enumerate((-1, 0, 1)):
            for dxi, dx in enumerate((-1, 0, 1)):
                s = dy * W + dx
                btile = lax.slice(ext, (0, PAD + s), (C, PAD + s + TN))
                m = masks[dxi]
                bts.append(btile if m is None else btile * m)

        def pass_dot(a):
            tap, chunk = a // 3, a % 3
            wv = w1_ref[tap, :, pl.ds(chunk * 256, 256)].astype(jnp.bfloat16)
            bv = lax.slice(bts[tap], (chunk * 256, 0), (chunk * 256 + 256, TN)).astype(jnp.bfloat16)
            return lax.dot_general(
                wv, bv, (((1,), (0,)), ((), ())),
                preferred_element_type=jnp.float32,
            )

        acc = None
        for lo, hi in GROUPS:
            g = pass_dot(lo)
            for a in range(lo + 1, hi):
                g = g + pass_dot(a)
            acc = g if acc is None else acc + g
        h = jnp.maximum(acc + b1_ref[...], 0.0)
        logit = lax.dot_general(
            w2_ref[...].astype(jnp.bfloat16), h.astype(jnp.bfloat16),
            (((1,), (0,)), ((), ())),
            preferred_element_type=jnp.float32,
        ) + b2_ref[0, 0]
        logit_ref[0, :, pl.ds(n0, TN)] = logit
        return 0

    lax.fori_loop(0, HW // TN, tile_body, 0)


def _topk_body(sc_ref, idx_ref, uu_ref, vv_ref):
    B, HW = sc_ref.shape
    W = 64
    K = idx_ref.shape[1]
    scores = sc_ref[...]
    iota = lax.broadcasted_iota(jnp.int32, (B, HW), 1)
    iota_k = lax.broadcasted_iota(jnp.int32, (B, K), 1)

    def body(t, carry):
        sc, acc = carry
        m = jnp.max(sc, axis=1, keepdims=True)
        cand = jnp.where(sc == m, iota, HW)
        idx = jnp.min(cand, axis=1, keepdims=True)
        acc = jnp.where(iota_k == t, idx, acc)
        return jnp.where(iota == idx, -jnp.inf, sc), acc

    _, idx = lax.fori_loop(0, K, body, (scores, jnp.zeros((B, K), jnp.int32)))
    idx_ref[...] = idx
    uu_ref[...] = (idx % W).astype(jnp.float32) / W
    vv_ref[...] = (idx // W).astype(jnp.float32) / W


def kernel(feat, W1, b1, W2, b2, num_anchors):
    B, C, H, W = feat.shape
    CM = W1.shape[0]
    HW = H * W
    PAD = 128
    K = 256

    featp = jnp.pad(feat.reshape(B, C, HW), ((0, 0), (0, 0), (PAD, PAD)))
    w1r = W1.transpose(2, 3, 0, 1).reshape(9, CM, C)
    b1r = b1.reshape(CM, 1)
    w2r = W2.reshape(1, CM)
    b2r = b2.reshape(1, 1)

    logits = pl.pallas_call(
        _conv_body,
        grid=(B,),
        in_specs=[
            pl.BlockSpec((1, C, HW + 2 * PAD), lambda b: (b, 0, 0)),
            pl.BlockSpec((9, CM, C), lambda b: (0, 0, 0)),
            pl.BlockSpec((CM, 1), lambda b: (0, 0)),
            pl.BlockSpec((1, CM), lambda b: (0, 0)),
            pl.BlockSpec((1, 1), lambda b: (0, 0)),
        ],
        out_specs=pl.BlockSpec((1, 1, HW), lambda b: (b, 0, 0)),
        out_shape=jax.ShapeDtypeStruct((B, 1, HW), jnp.float32),
    )(featp, w1r, b1r, w2r, b2r)

    heatmap = jax.nn.softplus(logits.reshape(B, 1, H, W))
    prob = heatmap.reshape(B, -1)
    prob = prob / (prob.sum(axis=1, keepdims=True) + 1e-06)
    gkey = jax.random.key(42)
    u = jax.random.uniform(gkey, prob.shape, minval=1e-10, maxval=1.0)
    gumbel = -jnp.log(-jnp.log(u))
    scores = jnp.log(prob + 1e-20) + gumbel

    idx, uu, vv = pl.pallas_call(
        _topk_body,
        out_shape=(
            jax.ShapeDtypeStruct((B, K), jnp.int32),
            jax.ShapeDtypeStruct((B, K), jnp.float32),
            jax.ShapeDtypeStruct((B, K), jnp.float32),
        ),
    )(scores)

    anchors = jnp.stack([uu, vv], axis=-1)
    return anchors, heatmap
